# baseline (speedup 1.0000x reference)
import functools

import jax
import jax.numpy as jnp
from jax import lax
from jax.experimental import pallas as pl

K_NEIGH = 32
H = 8

_MSB_INT = -2147483648
_HIGHEST = lax.Precision.HIGHEST


def _fold_body(C, d, Wtgt_ref, Wq_ref, WkT_ref, WsrcT_ref, Wsrc_ref, Wv_ref,
               Wo_ref, Wout_ref, btgt_ref, bsrc_ref, bout_ref,
               S2_ref, g_ref, F2_ref, c_ref):
    dh = d // H
    Wtq = jnp.dot(Wtgt_ref[...], Wq_ref[...], precision=_HIGHEST)
    WskT = jnp.dot(WkT_ref[...], WsrcT_ref[...], precision=_HIGHEST)
    Wsv = jnp.dot(Wsrc_ref[...], Wv_ref[...], precision=_HIGHEST)
    Wow = jnp.dot(Wo_ref[...], Wout_ref[...], precision=_HIGHEST)
    btq = jnp.dot(btgt_ref[...], Wq_ref[...], precision=_HIGHEST)
    bsv = jnp.dot(bsrc_ref[...], Wv_ref[...], precision=_HIGHEST)
    inv = 1.0 / jnp.sqrt(jnp.float32(dh))
    for h in range(H):
        hs = slice(h * dh, (h + 1) * dh)
        cs = slice(h * C, (h + 1) * C)
        S2_ref[:, cs] = jnp.dot(Wtq[:, hs], WskT[hs, :],
                                precision=_HIGHEST) * inv
        g_h = jnp.dot(btq[:, hs], WskT[hs, :], precision=_HIGHEST) * inv
        g_ref[:, cs] = jnp.broadcast_to(g_h, (8, C))
        F2_ref[cs, :] = jnp.dot(Wsv[:, hs], Wow[hs, :], precision=_HIGHEST)
    cvec = jnp.dot(bsv, Wow, precision=_HIGHEST) + bout_ref[...]
    c_ref[...] = jnp.broadcast_to(cvec, (8, C))


def _main_body(BN, N, C, A_ref, x0_ref, x2t_ref, x2o_ref,
               S2_ref, g_ref, F2_ref, c_ref, sq_ref, tret_ref):
    a = A_ref[0]
    x0 = x0_ref[0]
    x2t = x2t_ref[0]
    x2o = x2o_ref[0]

    third = jnp.float32(1.0 / 3.0)
    tgt = (x2t[:, 0:C] + x2t[:, C:2 * C] + x2t[:, 2 * C:3 * C]) * third
    tret_ref[0] = (x2o[0:C, :] + x2o[C:2 * C, :] + x2o[2 * C:3 * C, :]) * third

    u = jnp.dot(tgt, S2_ref[...], precision=_HIGHEST) + g_ref[0:1, :]

    msb = jnp.int32(_MSB_INT)
    bits = lax.bitcast_convert_type(a, jnp.int32)
    s = jnp.where(bits >= 0, bits, jnp.bitwise_xor(jnp.bitwise_not(bits), msb))
    def step(t, v):
        bit = jnp.left_shift(jnp.int32(1), 31 - t)
        cand = jnp.bitwise_or(v, bit)
        cnt = jnp.sum((s >= (cand ^ msb)).astype(jnp.int32), axis=1,
                      keepdims=True)
        return jnp.where(cnt >= K_NEIGH, cand, v)
    v = lax.fori_loop(0, 32, step, jnp.zeros((BN, 1), jnp.int32))
    mask = s >= (v ^ msb)

    neg = jnp.float32(-1e30)
    acc = c_ref[0:1, :]
    for h in range(H):
        cs = slice(h * C, (h + 1) * C)
        sc_h = lax.dot_general(u[:, cs], x0, (((1,), (1,)), ((), ())),
                               precision=None)
        masked = jnp.where(mask, sc_h, neg)
        mx = jnp.max(masked, axis=-1, keepdims=True)
        p = jnp.exp(masked - mx)
        denom = jnp.sum(p, axis=-1, keepdims=True)
        m_h = jnp.dot(p, x0) / denom
        acc = acc + jnp.dot(m_h, F2_ref[cs, :], precision=_HIGHEST)
    sq_ref[0] = acc


def _build(bs, N, C, d, BN, interpret=False):
    fold = pl.pallas_call(
        functools.partial(_fold_body, C, d),
        out_shape=(
            jax.ShapeDtypeStruct((C, H * C), jnp.float32),
            jax.ShapeDtypeStruct((8, H * C), jnp.float32),
            jax.ShapeDtypeStruct((H * C, C), jnp.float32),
            jax.ShapeDtypeStruct((8, C), jnp.float32),
        ),
        interpret=interpret,
    )

    grid = (bs, N // BN)
    main = pl.pallas_call(
        functools.partial(_main_body, BN, N, C),
        grid=grid,
        in_specs=[
            pl.BlockSpec((1, BN, N), lambda b, n: (b, n, 0)),
            pl.BlockSpec((1, N, C), lambda b, n: (b, 0, 0)),
            pl.BlockSpec((1, BN, 3 * C), lambda b, n: (b, n, 0)),
            pl.BlockSpec((1, 3 * C, BN), lambda b, n: (b, 0, n)),
            pl.BlockSpec((C, H * C), lambda b, n: (0, 0)),
            pl.BlockSpec((8, H * C), lambda b, n: (0, 0)),
            pl.BlockSpec((H * C, C), lambda b, n: (0, 0)),
            pl.BlockSpec((8, C), lambda b, n: (0, 0)),
        ],
        out_specs=(
            pl.BlockSpec((1, BN, C), lambda b, n: (b, n, 0)),
            pl.BlockSpec((1, 3 * C // 3, BN), lambda b, n: (b, 0, n)),
        ),
        out_shape=(
            jax.ShapeDtypeStruct((bs, N, C), jnp.float32),
            jax.ShapeDtypeStruct((bs, C, N), jnp.float32),
        ),
        interpret=interpret,
    )
    return fold, main


def kernel(x_c, x_c_2, mode, A, W_src, b_src, W_tgt, b_tgt,
           Wq, Wk, Wv, Wo, W_out, b_out):
    bs, C, _, N = x_c.shape
    d = W_src.shape[1]
    BN = 1024
    fold, main = _build(bs, N, C, d, BN)

    x0t = jnp.transpose(x_c[:, :, 0, :], (0, 2, 1))
    x2 = x_c_2[:, :, :, 0, :]
    x2o = x2.reshape(bs, 3 * C, N)
    x2t = jnp.transpose(x2, (0, 3, 1, 2)).reshape(bs, N, 3 * C)

    S2, g, F2, c = fold(W_tgt, Wq, Wk.T, W_src.T, W_src, Wv, Wo, W_out,
                        b_tgt.reshape(1, d), b_src.reshape(1, d),
                        b_out.reshape(1, C))
    sq_c, tgt_ret = main(A, x0t, x2t, x2o, S2, g, F2, c)
    return (sq_c, tgt_ret)


# speedup vs baseline: 1.0571x; 1.0571x over previous
import functools

import jax
import jax.numpy as jnp
from jax import lax
from jax.experimental import pallas as pl

K_NEIGH = 32
H = 8

_MSB_INT = -2147483648
_HIGHEST = lax.Precision.HIGHEST


def _fold_body(C, d, Wtgt_ref, Wq_ref, WkT_ref, WsrcT_ref, Wsrc_ref, Wv_ref,
               Wo_ref, Wout_ref, btgt_ref, bsrc_ref, bout_ref,
               S2_ref, g_ref, F2_ref, c_ref):
    dh = d // H
    Wtq = jnp.dot(Wtgt_ref[...], Wq_ref[...], precision=_HIGHEST)
    WskT = jnp.dot(WkT_ref[...], WsrcT_ref[...], precision=_HIGHEST)
    Wsv = jnp.dot(Wsrc_ref[...], Wv_ref[...], precision=_HIGHEST)
    Wow = jnp.dot(Wo_ref[...], Wout_ref[...], precision=_HIGHEST)
    btq = jnp.dot(btgt_ref[...], Wq_ref[...], precision=_HIGHEST)
    bsv = jnp.dot(bsrc_ref[...], Wv_ref[...], precision=_HIGHEST)
    inv = 1.0 / jnp.sqrt(jnp.float32(dh))
    for h in range(H):
        hs = slice(h * dh, (h + 1) * dh)
        cs = slice(h * C, (h + 1) * C)
        S2_ref[:, cs] = jnp.dot(Wtq[:, hs], WskT[hs, :],
                                precision=_HIGHEST) * inv
        g_h = jnp.dot(btq[:, hs], WskT[hs, :], precision=_HIGHEST) * inv
        g_ref[:, cs] = jnp.broadcast_to(g_h, (8, C))
        F2_ref[cs, :] = jnp.dot(Wsv[:, hs], Wow[hs, :], precision=_HIGHEST)
    cvec = jnp.dot(bsv, Wow, precision=_HIGHEST) + bout_ref[...]
    c_ref[...] = jnp.broadcast_to(cvec, (8, C))


def _main_body(BN, N, C, A_ref, x0_ref, x2t_ref, x2o_ref,
               S2_ref, g_ref, F2_ref, c_ref, sq_ref, tret_ref):
    a = A_ref[0]
    x0 = x0_ref[0]
    x2t = x2t_ref[0]
    x2o = x2o_ref[0]

    third = jnp.float32(1.0 / 3.0)
    tgt = (x2t[:, 0:C] + x2t[:, C:2 * C] + x2t[:, 2 * C:3 * C]) * third
    tret_ref[0] = (x2o[0:C, :] + x2o[C:2 * C, :] + x2o[2 * C:3 * C, :]) * third

    u = jnp.dot(tgt, S2_ref[...], precision=_HIGHEST) + g_ref[0:1, :]

    msb = jnp.int32(_MSB_INT)
    bits = lax.bitcast_convert_type(a, jnp.int32)
    s = jnp.where(bits >= 0, bits, jnp.bitwise_xor(jnp.bitwise_not(bits), msb))
    def step(t, v):
        bit = jnp.left_shift(jnp.int32(1), 31 - t)
        cand = jnp.bitwise_or(v, bit)
        cnt = jnp.sum((s >= (cand ^ msb)).astype(jnp.int32), axis=1,
                      keepdims=True)
        return jnp.where(cnt >= K_NEIGH, cand, v)
    v = lax.fori_loop(0, 32, step, jnp.zeros((BN, 1), jnp.int32))
    mask = s >= (v ^ msb)

    bias = jnp.where(mask, jnp.float32(0.0), jnp.float32(-1e30))
    acc = c_ref[0:1, :]
    for h in range(H):
        cs = slice(h * C, (h + 1) * C)
        sc_h = lax.dot_general(u[:, cs], x0, (((1,), (1,)), ((), ())),
                               precision=None)
        masked = sc_h + bias
        mx = jnp.max(masked, axis=-1, keepdims=True)
        p = jnp.exp(masked - mx)
        denom = jnp.sum(p, axis=-1, keepdims=True)
        m_h = jnp.dot(p, x0) / denom
        acc = acc + jnp.dot(m_h, F2_ref[cs, :], precision=_HIGHEST)
    sq_ref[0] = acc


def _build(bs, N, C, d, BN, interpret=False):
    fold = pl.pallas_call(
        functools.partial(_fold_body, C, d),
        out_shape=(
            jax.ShapeDtypeStruct((C, H * C), jnp.float32),
            jax.ShapeDtypeStruct((8, H * C), jnp.float32),
            jax.ShapeDtypeStruct((H * C, C), jnp.float32),
            jax.ShapeDtypeStruct((8, C), jnp.float32),
        ),
        interpret=interpret,
    )

    grid = (bs, N // BN)
    main = pl.pallas_call(
        functools.partial(_main_body, BN, N, C),
        grid=grid,
        in_specs=[
            pl.BlockSpec((1, BN, N), lambda b, n: (b, n, 0)),
            pl.BlockSpec((1, N, C), lambda b, n: (b, 0, 0)),
            pl.BlockSpec((1, BN, 3 * C), lambda b, n: (b, n, 0)),
            pl.BlockSpec((1, 3 * C, BN), lambda b, n: (b, 0, n)),
            pl.BlockSpec((C, H * C), lambda b, n: (0, 0)),
            pl.BlockSpec((8, H * C), lambda b, n: (0, 0)),
            pl.BlockSpec((H * C, C), lambda b, n: (0, 0)),
            pl.BlockSpec((8, C), lambda b, n: (0, 0)),
        ],
        out_specs=(
            pl.BlockSpec((1, BN, C), lambda b, n: (b, n, 0)),
            pl.BlockSpec((1, 3 * C // 3, BN), lambda b, n: (b, 0, n)),
        ),
        out_shape=(
            jax.ShapeDtypeStruct((bs, N, C), jnp.float32),
            jax.ShapeDtypeStruct((bs, C, N), jnp.float32),
        ),
        interpret=interpret,
    )
    return fold, main


def kernel(x_c, x_c_2, mode, A, W_src, b_src, W_tgt, b_tgt,
           Wq, Wk, Wv, Wo, W_out, b_out):
    bs, C, _, N = x_c.shape
    d = W_src.shape[1]
    BN = 512
    fold, main = _build(bs, N, C, d, BN)

    x0t = jnp.transpose(x_c[:, :, 0, :], (0, 2, 1))
    x2 = x_c_2[:, :, :, 0, :]
    x2o = x2.reshape(bs, 3 * C, N)
    x2t = jnp.transpose(x2, (0, 3, 1, 2)).reshape(bs, N, 3 * C)

    S2, g, F2, c = fold(W_tgt, Wq, Wk.T, W_src.T, W_src, Wv, Wo, W_out,
                        b_tgt.reshape(1, d), b_src.reshape(1, d),
                        b_out.reshape(1, C))
    sq_c, tgt_ret = main(A, x0t, x2t, x2o, S2, g, F2, c)
    return (sq_c, tgt_ret)
